# baseline (speedup 1.0000x reference)
import functools

import jax
import jax.numpy as jnp
from jax import lax
from jax.experimental import pallas as pl
from jax.experimental.pallas import tpu as pltpu
from jax.experimental.pallas import tpu_sc as plsc

N_NODES = 10000
N_PAD = 10240
E = 320000
D_IN = 128
D_HID = 128
D_OUT = 64

NC, NS = 2, 16
NW = NC * NS
E_PER_W = E // NW
CHUNK = 80
N_CHUNKS = E_PER_W // CHUNK
ROWS_PER_TILE = N_PAD // NS

_MESH = plsc.VectorSubcoreMesh(core_axis_name="c", subcore_axis_name="s")
_SC_PARAMS = pltpu.CompilerParams(use_tc_tiling_on_sc=False)


NB = 4
GA = 2
NFULL = (N_CHUNKS // NB) * NB


@functools.partial(
    pl.kernel,
    out_type=jax.ShapeDtypeStruct((NC, N_PAD, 16), jnp.float32),
    mesh=_MESH,
    scratch_types=(
        [pltpu.VMEM((CHUNK,), jnp.int32) for _ in range(NB)]
        + [pltpu.VMEM((CHUNK, 16), jnp.float32),
           pltpu.VMEM_SHARED((N_PAD, 16), jnp.float32)]
        + [pltpu.SemaphoreType.DMA for _ in range(NB)]
    ),
    compiler_params=_SC_PARAMS,
)
def _deg_kernel(edge_hbm, ones_hbm, zeros_hbm, out_hbm, *scr):
    cidx = scr[0:NB]
    ones_v, acc = scr[NB], scr[NB + 1]
    isem = scr[NB + 2:NB + 2 + NB]
    c = lax.axis_index("c")
    s = lax.axis_index("s")
    wid = c * NS + s
    e0 = wid * E_PER_W
    pltpu.sync_copy(zeros_hbm, acc.at[pl.ds(s * ROWS_PER_TILE, ROWS_PER_TILE)])
    pltpu.sync_copy(ones_hbm, ones_v)
    plsc.subcore_barrier()

    def cp_idx(k, q):
        return pltpu.async_copy(
            edge_hbm.at[1, pl.ds(e0 + k * CHUNK, CHUNK)], cidx[q], isem[q])

    def wait_idx(k, q):
        pltpu.make_async_copy(
            edge_hbm.at[1, pl.ds(e0 + k * CHUNK, CHUNK)],
            cidx[q], isem[q]).wait()

    for q in range(NB):
        cp_idx(q, q)

    def body(i0, carry):
        for q in range(NB):
            i = i0 * NB + q
            wait_idx(i, q)
            pltpu.sync_copy(ones_v, acc.at[cidx[q]], add=True)

            @pl.when(i + NB < N_CHUNKS)
            def _():
                cp_idx(i + NB, q)
        return carry

    lax.fori_loop(0, NFULL // NB, body, 0)
    for i in range(NFULL, N_CHUNKS):
        q = i % NB
        wait_idx(i, q)
        pltpu.sync_copy(ones_v, acc.at[cidx[q]], add=True)

    plsc.subcore_barrier()
    pltpu.sync_copy(
        acc.at[pl.ds(s * ROWS_PER_TILE, ROWS_PER_TILE)],
        out_hbm.at[c, pl.ds(s * ROWS_PER_TILE, ROWS_PER_TILE)],
    )


NI = 2 * NB

E_PER_W_PAD = 10240
NCH = E_PER_W_PAD // CHUNK
CHB = 16
NBLK = NCH // CHB


def _make_edge_scatter(d):
    @functools.partial(
        pl.kernel,
        out_type=jax.ShapeDtypeStruct((NC, N_PAD, d), jnp.float32),
        mesh=_MESH,
        scratch_types=(
            [pltpu.VMEM((CHB, 2, CHUNK), jnp.int32) for _ in range(2)]
            + [pltpu.VMEM((CHUNK, d), jnp.float32) for _ in range(NB)]
            + [pltpu.VMEM_SHARED((N_PAD, d), jnp.float32)]
            + [pltpu.SemaphoreType.DMA for _ in range(2 + 2 * NB)]
        ),
        compiler_params=_SC_PARAMS,
    )
    def edge_kernel(g_hbm, eidx_hbm, zeros_hbm, out_hbm, *scr):
        idxblk = scr[0:2]
        rows = scr[2:2 + NB]
        acc = scr[2 + NB]
        bsem = scr[3 + NB:5 + NB]
        gsem = scr[5 + NB:5 + 2 * NB]
        ssem = scr[5 + 2 * NB:5 + 3 * NB]
        c = lax.axis_index("c")
        s = lax.axis_index("s")
        wid = c * NS + s
        pltpu.sync_copy(zeros_hbm,
                        acc.at[pl.ds(s * ROWS_PER_TILE, ROWS_PER_TILE)])
        plsc.subcore_barrier()

        def cp_blk(bi, sl):
            return pltpu.async_copy(eidx_hbm.at[wid, bi], idxblk[sl],
                                    bsem[sl])

        def wait_blk(bi, sl):
            pltpu.make_async_copy(eidx_hbm.at[wid, bi], idxblk[sl],
                                  bsem[sl]).wait()

        def start_gather(sl, j, q):
            return pltpu.async_copy(g_hbm.at[idxblk[sl].at[j, 0]], rows[q],
                                    gsem[q])

        def wait_gather(sl, j, q):
            pltpu.make_async_copy(g_hbm.at[idxblk[sl].at[j, 0]], rows[q],
                                  gsem[q]).wait()

        def start_scatter(sl, j, q):
            return pltpu.async_copy(rows[q], acc.at[idxblk[sl].at[j, 1]],
                                    ssem[q], add=True)

        def wait_scatter(sl, j, q):
            pltpu.make_async_copy(rows[q], acc.at[idxblk[sl].at[j, 1]],
                                  ssem[q]).wait()

        def maybe(pred, fn):
            def run():
                fn()
            if isinstance(pred, bool):
                if pred:
                    run()
            else:
                pl.when(pred)(run)

        cp_blk(0, 0)
        wait_blk(0, 0)
        start_gather(0, 0, 0)
        start_gather(0, 1, 1)

        def body_one(bp, k):
            boff = k // CHB
            j = k % CHB
            q = k % NB
            sl = boff
            cc = bp * (2 * CHB) + k
            bb = bp * 2 + boff

            wait_gather(sl, j, q)
            start_scatter(sl, j, q)
            kp = (k - 2) % (2 * CHB)
            maybe(True if k >= 2 else cc >= 2,
                  lambda: wait_scatter(kp // CHB, kp % CHB, (q + 2) % NB))
            if j == 2:
                maybe(True if boff == 0 else bb + 1 < NBLK,
                      lambda: cp_blk(bb + 1, 1 - sl))

            def _gather_next():
                kn = (k + 2) % (2 * CHB)
                if j == 14:
                    wait_blk(bb + 1, 1 - sl)
                start_gather(kn // CHB, kn % CHB, (q + 2) % NB)
            maybe(cc + 2 < NCH if k >= 2 * CHB - 2 else True, _gather_next)

        def outer(bp, carry):
            for k in range(2 * CHB):
                body_one(bp, k)
            return carry

        lax.fori_loop(0, NBLK // 2, outer, 0)
        wait_scatter(1, CHB - 2, (NCH - 2) % NB)
        wait_scatter(1, CHB - 1, (NCH - 1) % NB)

        plsc.subcore_barrier()
        pltpu.sync_copy(
            acc.at[pl.ds(s * ROWS_PER_TILE, ROWS_PER_TILE)],
            out_hbm.at[c, pl.ds(s * ROWS_PER_TILE, ROWS_PER_TILE)],
        )

    return edge_kernel


_edge_scatter_128 = _make_edge_scatter(D_HID)
_edge_scatter_64 = _make_edge_scatter(D_OUT)


_BLK = 400


def _dis_block(degp_ref):
    deg = degp_ref[0, :, 0:1] + degp_ref[1, :, 0:1]
    return jnp.where(deg > 0, lax.rsqrt(deg), 0.0)


def _pre_body(x_ref, degp_ref, wi_ref, wr_ref, b_ref, g_ref, r_ref):
    dis = _dis_block(degp_ref)
    x = x_ref[...]
    h = jnp.dot(x, wi_ref[...], preferred_element_type=jnp.float32)
    g_ref[...] = h * dis
    r_ref[...] = (
        jnp.dot(x, wr_ref[...], preferred_element_type=jnp.float32)
        + b_ref[...]
    )


def _pre_kernel(x, deg_parts, wi, wr, b):
    d_in, d = wi.shape
    return pl.pallas_call(
        _pre_body,
        grid=(N_NODES // _BLK,),
        in_specs=[
            pl.BlockSpec((_BLK, d_in), lambda i: (i, 0)),
            pl.BlockSpec((NC, _BLK, 16), lambda i: (0, i, 0)),
            pl.BlockSpec((d_in, d), lambda i: (0, 0)),
            pl.BlockSpec((d_in, d), lambda i: (0, 0)),
            pl.BlockSpec((1, d), lambda i: (0, 0)),
        ],
        out_specs=[
            pl.BlockSpec((_BLK, d), lambda i: (i, 0)),
            pl.BlockSpec((_BLK, d), lambda i: (i, 0)),
        ],
        out_shape=[
            jax.ShapeDtypeStruct((N_NODES, d), jnp.float32),
            jax.ShapeDtypeStruct((N_NODES, d), jnp.float32),
        ],
    )(x, deg_parts, wi, wr, b)


def _mid_body(parts_ref, r_ref, degp_ref, wi_ref, wr_ref, b_ref,
              g_ref, rn_ref):
    dis = _dis_block(degp_ref)
    out = jnp.maximum(
        (parts_ref[0] + parts_ref[1]) * dis + r_ref[...], 0.0)
    h = jnp.dot(out, wi_ref[...], preferred_element_type=jnp.float32)
    g_ref[...] = h * dis
    rn_ref[...] = (
        jnp.dot(out, wr_ref[...], preferred_element_type=jnp.float32)
        + b_ref[...]
    )


def _mid_kernel(parts, r, deg_parts, wi, wr, b):
    d_in, d = wi.shape
    return pl.pallas_call(
        _mid_body,
        grid=(N_NODES // _BLK,),
        in_specs=[
            pl.BlockSpec((NC, _BLK, d_in), lambda i: (0, i, 0)),
            pl.BlockSpec((_BLK, d_in), lambda i: (i, 0)),
            pl.BlockSpec((NC, _BLK, 16), lambda i: (0, i, 0)),
            pl.BlockSpec((d_in, d), lambda i: (0, 0)),
            pl.BlockSpec((d_in, d), lambda i: (0, 0)),
            pl.BlockSpec((1, d), lambda i: (0, 0)),
        ],
        out_specs=[
            pl.BlockSpec((_BLK, d), lambda i: (i, 0)),
            pl.BlockSpec((_BLK, d), lambda i: (i, 0)),
        ],
        out_shape=[
            jax.ShapeDtypeStruct((N_NODES, d), jnp.float32),
            jax.ShapeDtypeStruct((N_NODES, d), jnp.float32),
        ],
    )(parts, r, deg_parts, wi, wr, b)


def _post_body(parts_ref, r_ref, degp_ref, out_ref):
    dis = _dis_block(degp_ref)
    z = (parts_ref[0] + parts_ref[1]) * dis + r_ref[...]
    out_ref[...] = jax.nn.sigmoid(jnp.maximum(z, 0.0))


def _post_kernel(parts, r, deg_parts):
    d = r.shape[1]
    return pl.pallas_call(
        _post_body,
        grid=(N_NODES // _BLK,),
        in_specs=[
            pl.BlockSpec((NC, _BLK, d), lambda i: (0, i, 0)),
            pl.BlockSpec((_BLK, d), lambda i: (i, 0)),
            pl.BlockSpec((NC, _BLK, 16), lambda i: (0, i, 0)),
        ],
        out_specs=pl.BlockSpec((_BLK, d), lambda i: (i, 0)),
        out_shape=jax.ShapeDtypeStruct((N_NODES, d), jnp.float32),
    )(parts, r, deg_parts)


def kernel(x, edge_index, batch, W1_init, W1_root, b1,
           W2_init, W2_root, b2, W3_init, W3_root, b3):
    zeros128 = jnp.zeros((ROWS_PER_TILE, D_HID), jnp.float32)
    zeros64 = jnp.zeros((ROWS_PER_TILE, D_OUT), jnp.float32)
    zeros16 = jnp.zeros((ROWS_PER_TILE, 16), jnp.float32)
    ones16 = jnp.ones((CHUNK, 16), jnp.float32)

    n_pad_edges = NW * E_PER_W_PAD - E
    pad = jnp.stack([
        jnp.zeros((n_pad_edges,), jnp.int32),
        jnp.full((n_pad_edges,), N_PAD - 1, jnp.int32),
    ])
    eidx = jnp.concatenate([edge_index, pad], axis=1)
    eidx = eidx.reshape(2, NW, NBLK, CHB, CHUNK).transpose(1, 2, 3, 0, 4)

    deg_parts = _deg_kernel(edge_index, ones16, zeros16)

    g, r = _pre_kernel(x, deg_parts, W1_init, W1_root,
                       jnp.reshape(b1, (1, -1)))
    parts = _edge_scatter_128(g, eidx, zeros128)
    g, r = _mid_kernel(parts, r, deg_parts, W2_init, W2_root,
                       jnp.reshape(b2, (1, -1)))
    parts = _edge_scatter_128(g, eidx, zeros128)
    g, r = _mid_kernel(parts, r, deg_parts, W3_init, W3_root,
                       jnp.reshape(b3, (1, -1)))
    parts = _edge_scatter_64(g, eidx, zeros64)
    return _post_kernel(parts, r, deg_parts)


# speedup vs baseline: 1.0315x; 1.0315x over previous
import functools

import jax
import jax.numpy as jnp
from jax import lax
from jax.experimental import pallas as pl
from jax.experimental.pallas import tpu as pltpu
from jax.experimental.pallas import tpu_sc as plsc

N_NODES = 10000
N_PAD = 10240
E = 320000
D_IN = 128
D_HID = 128
D_OUT = 64

NC, NS = 2, 16
NW = NC * NS
E_PER_W = E // NW
CHUNK = 80
N_CHUNKS = E_PER_W // CHUNK
ROWS_PER_TILE = N_PAD // NS

_MESH = plsc.VectorSubcoreMesh(core_axis_name="c", subcore_axis_name="s")
_SC_PARAMS = pltpu.CompilerParams(use_tc_tiling_on_sc=False)


NB = 4
GA = 2
NFULL = (N_CHUNKS // NB) * NB


@functools.partial(
    pl.kernel,
    out_type=jax.ShapeDtypeStruct((NC, N_PAD, 16), jnp.float32),
    mesh=_MESH,
    scratch_types=(
        [pltpu.VMEM((CHUNK,), jnp.int32) for _ in range(NB)]
        + [pltpu.VMEM((CHUNK, 16), jnp.float32),
           pltpu.VMEM_SHARED((N_PAD, 16), jnp.float32)]
        + [pltpu.SemaphoreType.DMA for _ in range(NB)]
    ),
    compiler_params=_SC_PARAMS,
)
def _deg_kernel(edge_hbm, ones_hbm, zeros_hbm, out_hbm, *scr):
    cidx = scr[0:NB]
    ones_v, acc = scr[NB], scr[NB + 1]
    isem = scr[NB + 2:NB + 2 + NB]
    c = lax.axis_index("c")
    s = lax.axis_index("s")
    wid = c * NS + s
    e0 = wid * E_PER_W
    pltpu.sync_copy(zeros_hbm, acc.at[pl.ds(s * ROWS_PER_TILE, ROWS_PER_TILE)])
    pltpu.sync_copy(ones_hbm, ones_v)
    plsc.subcore_barrier()

    def cp_idx(k, q):
        return pltpu.async_copy(
            edge_hbm.at[1, pl.ds(e0 + k * CHUNK, CHUNK)], cidx[q], isem[q])

    def wait_idx(k, q):
        pltpu.make_async_copy(
            edge_hbm.at[1, pl.ds(e0 + k * CHUNK, CHUNK)],
            cidx[q], isem[q]).wait()

    for q in range(NB):
        cp_idx(q, q)

    def body(i0, carry):
        for q in range(NB):
            i = i0 * NB + q
            wait_idx(i, q)
            pltpu.sync_copy(ones_v, acc.at[cidx[q]], add=True)

            @pl.when(i + NB < N_CHUNKS)
            def _():
                cp_idx(i + NB, q)
        return carry

    lax.fori_loop(0, NFULL // NB, body, 0)
    for i in range(NFULL, N_CHUNKS):
        q = i % NB
        wait_idx(i, q)
        pltpu.sync_copy(ones_v, acc.at[cidx[q]], add=True)

    plsc.subcore_barrier()
    pltpu.sync_copy(
        acc.at[pl.ds(s * ROWS_PER_TILE, ROWS_PER_TILE)],
        out_hbm.at[c, pl.ds(s * ROWS_PER_TILE, ROWS_PER_TILE)],
    )


NI = 2 * NB

E_PER_W_PAD = 10240
NCH = E_PER_W_PAD // CHUNK
CHB = 16
NBLK = NCH // CHB


def _make_edge_scatter(d):
    @functools.partial(
        pl.kernel,
        out_type=jax.ShapeDtypeStruct((NC, N_PAD, d), jnp.float32),
        mesh=_MESH,
        scratch_types=(
            [pltpu.VMEM((CHB, 2, CHUNK), jnp.int32) for _ in range(2)]
            + [pltpu.VMEM((CHUNK, d), jnp.float32) for _ in range(NB)]
            + [pltpu.VMEM_SHARED((N_PAD, d), jnp.float32)]
            + [pltpu.SemaphoreType.DMA for _ in range(2 + 2 * NB)]
        ),
        compiler_params=_SC_PARAMS,
    )
    def edge_kernel(g_hbm, eidx_hbm, zeros_hbm, out_hbm, *scr):
        idxblk = scr[0:2]
        rows = scr[2:2 + NB]
        acc = scr[2 + NB]
        bsem = scr[3 + NB:5 + NB]
        gsem = scr[5 + NB:5 + 2 * NB]
        ssem = scr[5 + 2 * NB:5 + 3 * NB]
        c = lax.axis_index("c")
        s = lax.axis_index("s")
        wid = c * NS + s
        pltpu.sync_copy(zeros_hbm,
                        acc.at[pl.ds(s * ROWS_PER_TILE, ROWS_PER_TILE)])
        plsc.subcore_barrier()

        def cp_blk(bi, sl):
            return pltpu.async_copy(eidx_hbm.at[wid, bi], idxblk[sl],
                                    bsem[sl])

        def wait_blk(bi, sl):
            pltpu.make_async_copy(eidx_hbm.at[wid, bi], idxblk[sl],
                                  bsem[sl]).wait()

        def start_gather(sl, j, q):
            return pltpu.async_copy(g_hbm.at[idxblk[sl].at[j, 0]], rows[q],
                                    gsem[q])

        def wait_gather(sl, j, q):
            pltpu.make_async_copy(g_hbm.at[idxblk[sl].at[j, 0]], rows[q],
                                  gsem[q]).wait()

        def start_scatter(sl, j, q):
            return pltpu.async_copy(rows[q], acc.at[idxblk[sl].at[j, 1]],
                                    ssem[q], add=True)

        def wait_scatter(sl, j, q):
            pltpu.make_async_copy(rows[q], acc.at[idxblk[sl].at[j, 1]],
                                  ssem[q]).wait()

        def maybe(pred, fn):
            def run():
                fn()
            if isinstance(pred, bool):
                if pred:
                    run()
            else:
                pl.when(pred)(run)

        cp_blk(0, 0)
        wait_blk(0, 0)
        start_gather(0, 0, 0)
        start_gather(0, 1, 1)

        def body_one(bp, k):
            boff = k // CHB
            j = k % CHB
            q = k % NB
            sl = boff
            cc = bp * (2 * CHB) + k
            bb = bp * 2 + boff

            wait_gather(sl, j, q)
            start_scatter(sl, j, q)
            kp = (k - 2) % (2 * CHB)
            maybe(True if k >= 2 else cc >= 2,
                  lambda: wait_scatter(kp // CHB, kp % CHB, (q + 2) % NB))
            if j == 2:
                maybe(True if boff == 0 else bb + 1 < NBLK,
                      lambda: cp_blk(bb + 1, 1 - sl))

            def _gather_next():
                kn = (k + 2) % (2 * CHB)
                if j == 14:
                    wait_blk(bb + 1, 1 - sl)
                start_gather(kn // CHB, kn % CHB, (q + 2) % NB)
            maybe(cc + 2 < NCH if k >= 2 * CHB - 2 else True, _gather_next)

        def outer(bp, carry):
            for k in range(2 * CHB):
                body_one(bp, k)
            return carry

        lax.fori_loop(0, NBLK // 2, outer, 0)
        wait_scatter(1, CHB - 2, (NCH - 2) % NB)
        wait_scatter(1, CHB - 1, (NCH - 1) % NB)

        plsc.subcore_barrier()
        pltpu.sync_copy(
            acc.at[pl.ds(s * ROWS_PER_TILE, ROWS_PER_TILE)],
            out_hbm.at[c, pl.ds(s * ROWS_PER_TILE, ROWS_PER_TILE)],
        )

    return edge_kernel


_edge_scatter_128 = _make_edge_scatter(D_HID)
_edge_scatter_64 = _make_edge_scatter(D_OUT)


_BLK = 400


def _dis_block(degp_ref):
    deg = degp_ref[0, :, 0:1] + degp_ref[1, :, 0:1]
    return jnp.where(deg > 0, lax.rsqrt(deg), 0.0)


def _pre_body(x_ref, degp_ref, wi_ref, wr_ref, b_ref, g_ref, r_ref):
    dis = _dis_block(degp_ref)
    x = x_ref[...]
    h = jnp.dot(x, wi_ref[...], preferred_element_type=jnp.float32)
    g_ref[...] = h * dis
    r_ref[...] = (
        jnp.dot(x, wr_ref[...], preferred_element_type=jnp.float32)
        + b_ref[...]
    )


def _pre_kernel(x, deg_parts, wi, wr, b):
    d_in, d = wi.shape
    return pl.pallas_call(
        _pre_body,
        grid=(N_NODES // _BLK,),
        in_specs=[
            pl.BlockSpec((_BLK, d_in), lambda i: (i, 0)),
            pl.BlockSpec((NC, _BLK, 16), lambda i: (0, i, 0)),
            pl.BlockSpec((d_in, d), lambda i: (0, 0)),
            pl.BlockSpec((d_in, d), lambda i: (0, 0)),
            pl.BlockSpec((1, d), lambda i: (0, 0)),
        ],
        out_specs=[
            pl.BlockSpec((_BLK, d), lambda i: (i, 0)),
            pl.BlockSpec((_BLK, d), lambda i: (i, 0)),
        ],
        out_shape=[
            jax.ShapeDtypeStruct((N_NODES, d), jnp.float32),
            jax.ShapeDtypeStruct((N_NODES, d), jnp.float32),
        ],
    )(x, deg_parts, wi, wr, b)


def _mid_body(parts_ref, r_ref, degp_ref, wi_ref, wr_ref, b_ref,
              g_ref, rn_ref):
    dis = _dis_block(degp_ref)
    out = jnp.maximum(
        (parts_ref[0] + parts_ref[1]) * dis + r_ref[...], 0.0)
    h = jnp.dot(out, wi_ref[...], preferred_element_type=jnp.float32)
    g_ref[...] = h * dis
    rn_ref[...] = (
        jnp.dot(out, wr_ref[...], preferred_element_type=jnp.float32)
        + b_ref[...]
    )


def _mid_kernel(parts, r, deg_parts, wi, wr, b):
    d_in, d = wi.shape
    return pl.pallas_call(
        _mid_body,
        grid=(N_NODES // _BLK,),
        in_specs=[
            pl.BlockSpec((NC, _BLK, d_in), lambda i: (0, i, 0)),
            pl.BlockSpec((_BLK, d_in), lambda i: (i, 0)),
            pl.BlockSpec((NC, _BLK, 16), lambda i: (0, i, 0)),
            pl.BlockSpec((d_in, d), lambda i: (0, 0)),
            pl.BlockSpec((d_in, d), lambda i: (0, 0)),
            pl.BlockSpec((1, d), lambda i: (0, 0)),
        ],
        out_specs=[
            pl.BlockSpec((_BLK, d), lambda i: (i, 0)),
            pl.BlockSpec((_BLK, d), lambda i: (i, 0)),
        ],
        out_shape=[
            jax.ShapeDtypeStruct((N_NODES, d), jnp.float32),
            jax.ShapeDtypeStruct((N_NODES, d), jnp.float32),
        ],
    )(parts, r, deg_parts, wi, wr, b)


def _post_body(parts_ref, r_ref, degp_ref, out_ref):
    dis = _dis_block(degp_ref)
    z = (parts_ref[0] + parts_ref[1]) * dis + r_ref[...]
    out_ref[...] = jax.nn.sigmoid(jnp.maximum(z, 0.0))


def _post_kernel(parts, r, deg_parts):
    d = r.shape[1]
    return pl.pallas_call(
        _post_body,
        grid=(N_NODES // _BLK,),
        in_specs=[
            pl.BlockSpec((NC, _BLK, d), lambda i: (0, i, 0)),
            pl.BlockSpec((_BLK, d), lambda i: (i, 0)),
            pl.BlockSpec((NC, _BLK, 16), lambda i: (0, i, 0)),
        ],
        out_specs=pl.BlockSpec((_BLK, d), lambda i: (i, 0)),
        out_shape=jax.ShapeDtypeStruct((N_NODES, d), jnp.float32),
    )(parts, r, deg_parts)


def kernel(x, edge_index, batch, W1_init, W1_root, b1,
           W2_init, W2_root, b2, W3_init, W3_root, b3):
    zeros128 = jnp.zeros((ROWS_PER_TILE, D_HID), jnp.float32)
    zeros64 = jnp.zeros((ROWS_PER_TILE, D_OUT), jnp.float32)
    zeros16 = jnp.zeros((ROWS_PER_TILE, 16), jnp.float32)
    ones16 = jnp.ones((CHUNK, 16), jnp.float32)

    n_pad_edges = NW * E_PER_W_PAD - E
    pad = jnp.stack([
        jnp.zeros((n_pad_edges,), jnp.int32),
        N_NODES + jnp.arange(n_pad_edges, dtype=jnp.int32)
        % (N_PAD - N_NODES),
    ])
    eidx = jnp.concatenate([edge_index, pad], axis=1)
    eidx = eidx.reshape(2, NW, NBLK, CHB, CHUNK).transpose(1, 2, 3, 0, 4)

    deg_parts = _deg_kernel(edge_index, ones16, zeros16)

    g, r = _pre_kernel(x, deg_parts, W1_init, W1_root,
                       jnp.reshape(b1, (1, -1)))
    parts = _edge_scatter_128(g, eidx, zeros128)
    g, r = _mid_kernel(parts, r, deg_parts, W2_init, W2_root,
                       jnp.reshape(b2, (1, -1)))
    parts = _edge_scatter_128(g, eidx, zeros128)
    g, r = _mid_kernel(parts, r, deg_parts, W3_init, W3_root,
                       jnp.reshape(b3, (1, -1)))
    parts = _edge_scatter_64(g, eidx, zeros64)
    return _post_kernel(parts, r, deg_parts)


# speedup vs baseline: 1.1284x; 1.0940x over previous
import functools

import jax
import jax.numpy as jnp
from jax import lax
from jax.experimental import pallas as pl
from jax.experimental.pallas import tpu as pltpu
from jax.experimental.pallas import tpu_sc as plsc

N_NODES = 10000
N_PAD = 10240
E = 320000
D_IN = 128
D_HID = 128
D_OUT = 64

NC, NS = 2, 16
NW = NC * NS
E_PER_W = E // NW
CHUNK = 80
N_CHUNKS = E_PER_W // CHUNK
ROWS_PER_TILE = N_PAD // NS

_MESH = plsc.VectorSubcoreMesh(core_axis_name="c", subcore_axis_name="s")
_SC_PARAMS = pltpu.CompilerParams(use_tc_tiling_on_sc=False)


NB = 4
GA = 2
NFULL = (N_CHUNKS // NB) * NB


@functools.partial(
    pl.kernel,
    out_type=jax.ShapeDtypeStruct((NC, N_PAD, 16), jnp.float32),
    mesh=_MESH,
    scratch_types=(
        [pltpu.VMEM((CHUNK,), jnp.int32) for _ in range(NB)]
        + [pltpu.VMEM((CHUNK, 16), jnp.float32),
           pltpu.VMEM_SHARED((N_PAD, 16), jnp.float32)]
        + [pltpu.SemaphoreType.DMA for _ in range(NB)]
    ),
    compiler_params=_SC_PARAMS,
)
def _deg_kernel(edge_hbm, ones_hbm, zeros_hbm, out_hbm, *scr):
    cidx = scr[0:NB]
    ones_v, acc = scr[NB], scr[NB + 1]
    isem = scr[NB + 2:NB + 2 + NB]
    c = lax.axis_index("c")
    s = lax.axis_index("s")
    wid = c * NS + s
    e0 = wid * E_PER_W
    pltpu.sync_copy(zeros_hbm, acc.at[pl.ds(s * ROWS_PER_TILE, ROWS_PER_TILE)])
    pltpu.sync_copy(ones_hbm, ones_v)
    plsc.subcore_barrier()

    def cp_idx(k, q):
        return pltpu.async_copy(
            edge_hbm.at[1, pl.ds(e0 + k * CHUNK, CHUNK)], cidx[q], isem[q])

    def wait_idx(k, q):
        pltpu.make_async_copy(
            edge_hbm.at[1, pl.ds(e0 + k * CHUNK, CHUNK)],
            cidx[q], isem[q]).wait()

    for q in range(NB):
        cp_idx(q, q)

    def body(i0, carry):
        for q in range(NB):
            i = i0 * NB + q
            wait_idx(i, q)
            pltpu.sync_copy(ones_v, acc.at[cidx[q]], add=True)

            @pl.when(i + NB < N_CHUNKS)
            def _():
                cp_idx(i + NB, q)
        return carry

    lax.fori_loop(0, NFULL // NB, body, 0)
    for i in range(NFULL, N_CHUNKS):
        q = i % NB
        wait_idx(i, q)
        pltpu.sync_copy(ones_v, acc.at[cidx[q]], add=True)

    plsc.subcore_barrier()
    pltpu.sync_copy(
        acc.at[pl.ds(s * ROWS_PER_TILE, ROWS_PER_TILE)],
        out_hbm.at[c, pl.ds(s * ROWS_PER_TILE, ROWS_PER_TILE)],
    )


NI = 2 * NB

E_PER_W_PAD = 10240
NCH = E_PER_W_PAD // CHUNK
CHB = 16
NBLK = NCH // CHB


def _make_edge_scatter(d):
    @functools.partial(
        pl.kernel,
        out_type=jax.ShapeDtypeStruct((NC, N_PAD, d), jnp.float32),
        mesh=_MESH,
        scratch_types=(
            [pltpu.VMEM((CHB, 2, CHUNK), jnp.int32) for _ in range(2)]
            + [pltpu.VMEM((CHUNK, d), jnp.float32) for _ in range(NB)]
            + [pltpu.VMEM_SHARED((N_PAD, d), jnp.float32)]
            + [pltpu.SemaphoreType.DMA for _ in range(2 + 2 * NB)]
        ),
        compiler_params=_SC_PARAMS,
    )
    def edge_kernel(g_hbm, eidx_hbm, zeros_hbm, out_hbm, *scr):
        idxblk = scr[0:2]
        rows = scr[2:2 + NB]
        acc = scr[2 + NB]
        bsem = scr[3 + NB:5 + NB]
        gsem = scr[5 + NB:5 + 2 * NB]
        ssem = scr[5 + 2 * NB:5 + 3 * NB]
        c = lax.axis_index("c")
        s = lax.axis_index("s")
        wid = c * NS + s
        pltpu.sync_copy(zeros_hbm,
                        acc.at[pl.ds(s * ROWS_PER_TILE, ROWS_PER_TILE)])
        plsc.subcore_barrier()

        def cp_blk(bi, sl):
            return pltpu.async_copy(eidx_hbm.at[wid, bi], idxblk[sl],
                                    bsem[sl])

        def wait_blk(bi, sl):
            pltpu.make_async_copy(eidx_hbm.at[wid, bi], idxblk[sl],
                                  bsem[sl]).wait()

        def start_gather(sl, j, q):
            return pltpu.async_copy(g_hbm.at[idxblk[sl].at[j, 0]], rows[q],
                                    gsem[q])

        def wait_gather(sl, j, q):
            pltpu.make_async_copy(g_hbm.at[idxblk[sl].at[j, 0]], rows[q],
                                  gsem[q]).wait()

        def start_scatter(sl, j, q):
            return pltpu.async_copy(rows[q], acc.at[idxblk[sl].at[j, 1]],
                                    ssem[q], add=True)

        def wait_scatter(sl, j, q):
            pltpu.make_async_copy(rows[q], acc.at[idxblk[sl].at[j, 1]],
                                  ssem[q]).wait()

        def maybe(pred, fn):
            def run():
                fn()
            if isinstance(pred, bool):
                if pred:
                    run()
            else:
                pl.when(pred)(run)

        cp_blk(0, 0)
        wait_blk(0, 0)
        start_gather(0, 0, 0)
        start_gather(0, 1, 1)

        def body_one(bp, k):
            boff = k // CHB
            j = k % CHB
            q = k % NB
            sl = boff
            cc = bp * (2 * CHB) + k
            bb = bp * 2 + boff

            wait_gather(sl, j, q)
            start_scatter(sl, j, q)
            kp = (k - 2) % (2 * CHB)
            maybe(True if k >= 2 else cc >= 2,
                  lambda: wait_scatter(kp // CHB, kp % CHB, (q + 2) % NB))
            if j == 2:
                maybe(True if boff == 0 else bb + 1 < NBLK,
                      lambda: cp_blk(bb + 1, 1 - sl))

            def _gather_next():
                kn = (k + 2) % (2 * CHB)
                if j == 14:
                    wait_blk(bb + 1, 1 - sl)
                start_gather(kn // CHB, kn % CHB, (q + 2) % NB)
            maybe(cc + 2 < NCH if k >= 2 * CHB - 2 else True, _gather_next)

        def outer(bp, carry):
            for k in range(2 * CHB):
                body_one(bp, k)
            return carry

        lax.fori_loop(0, NBLK // 2, outer, 0)
        wait_scatter(1, CHB - 2, (NCH - 2) % NB)
        wait_scatter(1, CHB - 1, (NCH - 1) % NB)

        plsc.subcore_barrier()
        pltpu.sync_copy(
            acc.at[pl.ds(s * ROWS_PER_TILE, ROWS_PER_TILE)],
            out_hbm.at[c, pl.ds(s * ROWS_PER_TILE, ROWS_PER_TILE)],
        )

    return edge_kernel


_edge_scatter_128 = _make_edge_scatter(D_HID)
_edge_scatter_64 = _make_edge_scatter(D_OUT)


_BLK = 400


def _dis_block(degp_ref):
    deg = degp_ref[0, :, 0:1] + degp_ref[1, :, 0:1]
    return jnp.where(deg > 0, lax.rsqrt(deg), 0.0)


def _pre_body(x_ref, degp_ref, wi_ref, wr_ref, b_ref, g_ref, r_ref):
    dis = _dis_block(degp_ref)
    x = x_ref[...]
    h = jnp.dot(x, wi_ref[...], preferred_element_type=jnp.float32)
    g_ref[...] = h * dis
    r_ref[...] = (
        jnp.dot(x, wr_ref[...], preferred_element_type=jnp.float32)
        + b_ref[...]
    )


def _pre_kernel(x, deg_parts, wi, wr, b):
    d_in, d = wi.shape
    return pl.pallas_call(
        _pre_body,
        grid=(N_NODES // _BLK,),
        in_specs=[
            pl.BlockSpec((_BLK, d_in), lambda i: (i, 0)),
            pl.BlockSpec((NC, _BLK, 16), lambda i: (0, i, 0)),
            pl.BlockSpec((d_in, d), lambda i: (0, 0)),
            pl.BlockSpec((d_in, d), lambda i: (0, 0)),
            pl.BlockSpec((1, d), lambda i: (0, 0)),
        ],
        out_specs=[
            pl.BlockSpec((_BLK, d), lambda i: (i, 0)),
            pl.BlockSpec((_BLK, d), lambda i: (i, 0)),
        ],
        out_shape=[
            jax.ShapeDtypeStruct((N_NODES, d), jnp.float32),
            jax.ShapeDtypeStruct((N_NODES, d), jnp.float32),
        ],
    )(x, deg_parts, wi, wr, b)


def _mid_body(parts_ref, r_ref, degp_ref, wi_ref, wr_ref, b_ref,
              g_ref, rn_ref):
    dis = _dis_block(degp_ref)
    out = jnp.maximum(
        (parts_ref[0] + parts_ref[1]) * dis + r_ref[...], 0.0)
    h = jnp.dot(out, wi_ref[...], preferred_element_type=jnp.float32)
    g_ref[...] = h * dis
    rn_ref[...] = (
        jnp.dot(out, wr_ref[...], preferred_element_type=jnp.float32)
        + b_ref[...]
    )


def _mid_kernel(parts, r, deg_parts, wi, wr, b):
    d_in, d = wi.shape
    return pl.pallas_call(
        _mid_body,
        grid=(N_NODES // _BLK,),
        in_specs=[
            pl.BlockSpec((NC, _BLK, d_in), lambda i: (0, i, 0)),
            pl.BlockSpec((_BLK, d_in), lambda i: (i, 0)),
            pl.BlockSpec((NC, _BLK, 16), lambda i: (0, i, 0)),
            pl.BlockSpec((d_in, d), lambda i: (0, 0)),
            pl.BlockSpec((d_in, d), lambda i: (0, 0)),
            pl.BlockSpec((1, d), lambda i: (0, 0)),
        ],
        out_specs=[
            pl.BlockSpec((_BLK, d), lambda i: (i, 0)),
            pl.BlockSpec((_BLK, d), lambda i: (i, 0)),
        ],
        out_shape=[
            jax.ShapeDtypeStruct((N_NODES, d), jnp.float32),
            jax.ShapeDtypeStruct((N_NODES, d), jnp.float32),
        ],
    )(parts, r, deg_parts, wi, wr, b)


def _post_body(parts_ref, r_ref, degp_ref, out_ref):
    dis = _dis_block(degp_ref)
    z = (parts_ref[0] + parts_ref[1]) * dis + r_ref[...]
    out_ref[...] = jax.nn.sigmoid(jnp.maximum(z, 0.0))


def _post_kernel(parts, r, deg_parts):
    d = r.shape[1]
    return pl.pallas_call(
        _post_body,
        grid=(N_NODES // _BLK,),
        in_specs=[
            pl.BlockSpec((NC, _BLK, d), lambda i: (0, i, 0)),
            pl.BlockSpec((_BLK, d), lambda i: (i, 0)),
            pl.BlockSpec((NC, _BLK, 16), lambda i: (0, i, 0)),
        ],
        out_specs=pl.BlockSpec((_BLK, d), lambda i: (i, 0)),
        out_shape=jax.ShapeDtypeStruct((N_NODES, d), jnp.float32),
    )(parts, r, deg_parts)


def kernel(x, edge_index, batch, W1_init, W1_root, b1,
           W2_init, W2_root, b2, W3_init, W3_root, b3):
    zeros128 = jnp.zeros((ROWS_PER_TILE, D_HID), jnp.float32)
    zeros64 = jnp.zeros((ROWS_PER_TILE, D_OUT), jnp.float32)
    zeros16 = jnp.zeros((ROWS_PER_TILE, 16), jnp.float32)
    ones16 = jnp.ones((CHUNK, 16), jnp.float32)

    n_tile_pad = E_PER_W_PAD - E_PER_W
    pad = jnp.stack([
        jnp.zeros((NW, n_tile_pad), jnp.int32),
        jnp.broadcast_to(
            N_NODES + jnp.arange(n_tile_pad, dtype=jnp.int32)[None, :],
            (NW, n_tile_pad)),
    ])
    eidx = jnp.concatenate([edge_index.reshape(2, NW, E_PER_W), pad],
                           axis=2)
    eidx = eidx.reshape(2, NW, NBLK, CHB, CHUNK).transpose(1, 2, 3, 0, 4)

    deg_parts = _deg_kernel(edge_index, ones16, zeros16)

    g, r = _pre_kernel(x, deg_parts, W1_init, W1_root,
                       jnp.reshape(b1, (1, -1)))
    parts = _edge_scatter_128(g, eidx, zeros128)
    g, r = _mid_kernel(parts, r, deg_parts, W2_init, W2_root,
                       jnp.reshape(b2, (1, -1)))
    parts = _edge_scatter_128(g, eidx, zeros128)
    g, r = _mid_kernel(parts, r, deg_parts, W3_init, W3_root,
                       jnp.reshape(b3, (1, -1)))
    parts = _edge_scatter_64(g, eidx, zeros64)
    return _post_kernel(parts, r, deg_parts)


# speedup vs baseline: 3.1784x; 2.8167x over previous
import functools

import jax
import jax.numpy as jnp
from jax import lax
from jax.experimental import pallas as pl
from jax.experimental.pallas import tpu as pltpu
from jax.experimental.pallas import tpu_sc as plsc

N_NODES = 10000
N_PAD = 10240
E = 320000
D_IN = 128
D_HID = 128
D_OUT = 64

NC, NS = 2, 16
NW = NC * NS
E_PER_W = E // NW
CHUNK = 80
N_CHUNKS = E_PER_W // CHUNK
ROWS_PER_TILE = N_PAD // NS

_MESH = plsc.VectorSubcoreMesh(core_axis_name="c", subcore_axis_name="s")
_SC_PARAMS = pltpu.CompilerParams(use_tc_tiling_on_sc=False)


NB = 4
GA = 3
NFULL = (N_CHUNKS // NB) * NB


@functools.partial(
    pl.kernel,
    out_type=jax.ShapeDtypeStruct((NC, N_PAD, 16), jnp.float32),
    mesh=_MESH,
    scratch_types=(
        [pltpu.VMEM((CHUNK,), jnp.int32) for _ in range(NB)]
        + [pltpu.VMEM((CHUNK, 16), jnp.float32),
           pltpu.VMEM_SHARED((N_PAD, 16), jnp.float32)]
        + [pltpu.SemaphoreType.DMA for _ in range(NB)]
    ),
    compiler_params=_SC_PARAMS,
)
def _deg_kernel(edge_hbm, ones_hbm, zeros_hbm, out_hbm, *scr):
    cidx = scr[0:NB]
    ones_v, acc = scr[NB], scr[NB + 1]
    isem = scr[NB + 2:NB + 2 + NB]
    c = lax.axis_index("c")
    s = lax.axis_index("s")
    wid = c * NS + s
    e0 = wid * E_PER_W
    pltpu.sync_copy(zeros_hbm, acc.at[pl.ds(s * ROWS_PER_TILE, ROWS_PER_TILE)])
    pltpu.sync_copy(ones_hbm, ones_v)
    plsc.subcore_barrier()

    def cp_idx(k, q):
        return pltpu.async_copy(
            edge_hbm.at[1, pl.ds(e0 + k * CHUNK, CHUNK)], cidx[q], isem[q])

    def wait_idx(k, q):
        pltpu.make_async_copy(
            edge_hbm.at[1, pl.ds(e0 + k * CHUNK, CHUNK)],
            cidx[q], isem[q]).wait()

    for q in range(NB):
        cp_idx(q, q)

    def body(i0, carry):
        for q in range(NB):
            i = i0 * NB + q
            wait_idx(i, q)
            pltpu.sync_copy(ones_v, acc.at[cidx[q]], add=True)

            @pl.when(i + NB < N_CHUNKS)
            def _():
                cp_idx(i + NB, q)
        return carry

    lax.fori_loop(0, NFULL // NB, body, 0)
    for i in range(NFULL, N_CHUNKS):
        q = i % NB
        wait_idx(i, q)
        pltpu.sync_copy(ones_v, acc.at[cidx[q]], add=True)

    plsc.subcore_barrier()
    pltpu.sync_copy(
        acc.at[pl.ds(s * ROWS_PER_TILE, ROWS_PER_TILE)],
        out_hbm.at[c, pl.ds(s * ROWS_PER_TILE, ROWS_PER_TILE)],
    )


NI = 2 * NB


def _make_edge_scatter(d):
    @functools.partial(
        pl.kernel,
        out_type=jax.ShapeDtypeStruct((NC, N_PAD, d), jnp.float32),
        mesh=_MESH,
        scratch_types=(
            [pltpu.VMEM((2, CHUNK), jnp.int32) for _ in range(NI)]
            + [pltpu.VMEM((CHUNK, d), jnp.float32) for _ in range(NB)]
            + [pltpu.VMEM_SHARED((N_PAD, d), jnp.float32)]
            + [pltpu.SemaphoreType.DMA for _ in range(NI + 2 * NB)]
        ),
        compiler_params=_SC_PARAMS,
    )
    def edge_kernel(g_hbm, edge_hbm, zeros_hbm, out_hbm, *scr):
        idxb = scr[0:NI]
        rows = scr[NI:NI + NB]
        acc = scr[NI + NB]
        isem = scr[NI + NB + 1:2 * NI + NB + 1]
        gsem = scr[2 * NI + NB + 1:2 * NI + 2 * NB + 1]
        ssem = scr[2 * NI + 2 * NB + 1:2 * NI + 3 * NB + 1]
        c = lax.axis_index("c")
        s = lax.axis_index("s")
        wid = c * NS + s
        e0 = wid * E_PER_W
        pltpu.sync_copy(zeros_hbm,
                        acc.at[pl.ds(s * ROWS_PER_TILE, ROWS_PER_TILE)])
        plsc.subcore_barrier()

        def cp_idx(k, si):
            return pltpu.async_copy(
                edge_hbm.at[:, pl.ds(e0 + k * CHUNK, CHUNK)],
                idxb[si], isem[si])

        def wait_idx(k, si):
            pltpu.make_async_copy(
                edge_hbm.at[:, pl.ds(e0 + k * CHUNK, CHUNK)],
                idxb[si], isem[si]).wait()

        def start_gather(si, q):
            return pltpu.async_copy(g_hbm.at[idxb[si].at[0]], rows[q],
                                    gsem[q])

        def wait_gather(si, q):
            pltpu.make_async_copy(g_hbm.at[idxb[si].at[0]], rows[q],
                                  gsem[q]).wait()

        def start_scatter(si, q):
            return pltpu.async_copy(rows[q], acc.at[idxb[si].at[1]],
                                    ssem[q], add=True)

        def wait_scatter(si, q):
            pltpu.make_async_copy(rows[q], acc.at[idxb[si].at[1]],
                                  ssem[q]).wait()

        def maybe(pred, fn):
            def run():
                fn()
            if isinstance(pred, bool):
                if pred:
                    run()
            else:
                pl.when(pred)(run)

        for si in range(NI - GA):
            cp_idx(si, si)
        for k in range(GA):
            wait_idx(k, k)
            start_gather(k, k)

        def body_one(i, q, si):
            wait_gather(si, q)
            start_scatter(si, q)
            qn = (q + GA) % NB
            sn = (si - (NB - GA)) % NI
            maybe(i >= NB - GA, lambda: wait_scatter(sn, qn))
            maybe(i + NI - GA < N_CHUNKS,
                  lambda: cp_idx(i + NI - GA, (si - GA) % NI))

            def _gather_next():
                wait_idx(i + GA, (si + GA) % NI)
                start_gather((si + GA) % NI, qn)
            maybe(i + GA < N_CHUNKS, _gather_next)

        def outer(j0, carry):
            for k in range(NI):
                body_one(j0 * NI + k, k % NB, k % NI)
            return carry

        NOUTER = N_CHUNKS // NI
        lax.fori_loop(0, NOUTER, outer, 0)
        for i in range(NOUTER * NI, N_CHUNKS):
            body_one(i, i % NB, i % NI)
        for i in range(N_CHUNKS - (NB - GA), N_CHUNKS):
            wait_scatter(i % NI, i % NB)

        plsc.subcore_barrier()
        pltpu.sync_copy(
            acc.at[pl.ds(s * ROWS_PER_TILE, ROWS_PER_TILE)],
            out_hbm.at[c, pl.ds(s * ROWS_PER_TILE, ROWS_PER_TILE)],
        )

    return edge_kernel


_edge_scatter_128 = _make_edge_scatter(D_HID)
_edge_scatter_64 = _make_edge_scatter(D_OUT)


_BLK = 400


def _dis_block(degp_ref):
    deg = degp_ref[0, :, 0:1] + degp_ref[1, :, 0:1]
    return jnp.where(deg > 0, lax.rsqrt(deg), 0.0)


def _pre_body(x_ref, degp_ref, wi_ref, wr_ref, b_ref, g_ref, r_ref):
    dis = _dis_block(degp_ref)
    x = x_ref[...]
    h = jnp.dot(x, wi_ref[...], preferred_element_type=jnp.float32)
    g_ref[...] = h * dis
    r_ref[...] = (
        jnp.dot(x, wr_ref[...], preferred_element_type=jnp.float32)
        + b_ref[...]
    )


def _pre_kernel(x, deg_parts, wi, wr, b):
    d_in, d = wi.shape
    return pl.pallas_call(
        _pre_body,
        grid=(N_NODES // _BLK,),
        in_specs=[
            pl.BlockSpec((_BLK, d_in), lambda i: (i, 0)),
            pl.BlockSpec((NC, _BLK, 16), lambda i: (0, i, 0)),
            pl.BlockSpec((d_in, d), lambda i: (0, 0)),
            pl.BlockSpec((d_in, d), lambda i: (0, 0)),
            pl.BlockSpec((1, d), lambda i: (0, 0)),
        ],
        out_specs=[
            pl.BlockSpec((_BLK, d), lambda i: (i, 0)),
            pl.BlockSpec((_BLK, d), lambda i: (i, 0)),
        ],
        out_shape=[
            jax.ShapeDtypeStruct((N_NODES, d), jnp.float32),
            jax.ShapeDtypeStruct((N_NODES, d), jnp.float32),
        ],
    )(x, deg_parts, wi, wr, b)


def _mid_body(parts_ref, r_ref, degp_ref, wi_ref, wr_ref, b_ref,
              g_ref, rn_ref):
    dis = _dis_block(degp_ref)
    out = jnp.maximum(
        (parts_ref[0] + parts_ref[1]) * dis + r_ref[...], 0.0)
    h = jnp.dot(out, wi_ref[...], preferred_element_type=jnp.float32)
    g_ref[...] = h * dis
    rn_ref[...] = (
        jnp.dot(out, wr_ref[...], preferred_element_type=jnp.float32)
        + b_ref[...]
    )


def _mid_kernel(parts, r, deg_parts, wi, wr, b):
    d_in, d = wi.shape
    return pl.pallas_call(
        _mid_body,
        grid=(N_NODES // _BLK,),
        in_specs=[
            pl.BlockSpec((NC, _BLK, d_in), lambda i: (0, i, 0)),
            pl.BlockSpec((_BLK, d_in), lambda i: (i, 0)),
            pl.BlockSpec((NC, _BLK, 16), lambda i: (0, i, 0)),
            pl.BlockSpec((d_in, d), lambda i: (0, 0)),
            pl.BlockSpec((d_in, d), lambda i: (0, 0)),
            pl.BlockSpec((1, d), lambda i: (0, 0)),
        ],
        out_specs=[
            pl.BlockSpec((_BLK, d), lambda i: (i, 0)),
            pl.BlockSpec((_BLK, d), lambda i: (i, 0)),
        ],
        out_shape=[
            jax.ShapeDtypeStruct((N_NODES, d), jnp.float32),
            jax.ShapeDtypeStruct((N_NODES, d), jnp.float32),
        ],
    )(parts, r, deg_parts, wi, wr, b)


def _post_body(parts_ref, r_ref, degp_ref, out_ref):
    dis = _dis_block(degp_ref)
    z = (parts_ref[0] + parts_ref[1]) * dis + r_ref[...]
    out_ref[...] = jax.nn.sigmoid(jnp.maximum(z, 0.0))


def _post_kernel(parts, r, deg_parts):
    d = r.shape[1]
    return pl.pallas_call(
        _post_body,
        grid=(N_NODES // _BLK,),
        in_specs=[
            pl.BlockSpec((NC, _BLK, d), lambda i: (0, i, 0)),
            pl.BlockSpec((_BLK, d), lambda i: (i, 0)),
            pl.BlockSpec((NC, _BLK, 16), lambda i: (0, i, 0)),
        ],
        out_specs=pl.BlockSpec((_BLK, d), lambda i: (i, 0)),
        out_shape=jax.ShapeDtypeStruct((N_NODES, d), jnp.float32),
    )(parts, r, deg_parts)


def kernel(x, edge_index, batch, W1_init, W1_root, b1,
           W2_init, W2_root, b2, W3_init, W3_root, b3):
    zeros128 = jnp.zeros((ROWS_PER_TILE, D_HID), jnp.float32)
    zeros64 = jnp.zeros((ROWS_PER_TILE, D_OUT), jnp.float32)
    zeros16 = jnp.zeros((ROWS_PER_TILE, 16), jnp.float32)
    ones16 = jnp.ones((CHUNK, 16), jnp.float32)

    deg_parts = _deg_kernel(edge_index, ones16, zeros16)

    g, r = _pre_kernel(x, deg_parts, W1_init, W1_root,
                       jnp.reshape(b1, (1, -1)))
    parts = _edge_scatter_128(g, edge_index, zeros128)
    g, r = _mid_kernel(parts, r, deg_parts, W2_init, W2_root,
                       jnp.reshape(b2, (1, -1)))
    parts = _edge_scatter_128(g, edge_index, zeros128)
    g, r = _mid_kernel(parts, r, deg_parts, W3_init, W3_root,
                       jnp.reshape(b3, (1, -1)))
    parts = _edge_scatter_64(g, edge_index, zeros64)
    return _post_kernel(parts, r, deg_parts)


# speedup vs baseline: 3.1844x; 1.0019x over previous
import functools

import jax
import jax.numpy as jnp
from jax import lax
from jax.experimental import pallas as pl
from jax.experimental.pallas import tpu as pltpu
from jax.experimental.pallas import tpu_sc as plsc

N_NODES = 10000
N_PAD = 10240
E = 320000
D_IN = 128
D_HID = 128
D_OUT = 64

NC, NS = 2, 16
NW = NC * NS
E_PER_W = E // NW
CHUNK = 80
N_CHUNKS = E_PER_W // CHUNK
ROWS_PER_TILE = N_PAD // NS

_MESH = plsc.VectorSubcoreMesh(core_axis_name="c", subcore_axis_name="s")
_SC_PARAMS = pltpu.CompilerParams(use_tc_tiling_on_sc=False)


NB = 4
GA = 3
NFULL = (N_CHUNKS // NB) * NB


@functools.partial(
    pl.kernel,
    out_type=jax.ShapeDtypeStruct((NC, N_PAD, 16), jnp.float32),
    mesh=_MESH,
    scratch_types=(
        [pltpu.VMEM((CHUNK,), jnp.int32) for _ in range(NB)]
        + [pltpu.VMEM((CHUNK, 16), jnp.float32),
           pltpu.VMEM_SHARED((N_PAD, 16), jnp.float32)]
        + [pltpu.SemaphoreType.DMA for _ in range(NB)]
    ),
    compiler_params=_SC_PARAMS,
)
def _deg_kernel(edge_hbm, ones_hbm, zeros_hbm, out_hbm, *scr):
    cidx = scr[0:NB]
    ones_v, acc = scr[NB], scr[NB + 1]
    isem = scr[NB + 2:NB + 2 + NB]
    c = lax.axis_index("c")
    s = lax.axis_index("s")
    wid = c * NS + s
    e0 = wid * E_PER_W
    pltpu.sync_copy(zeros_hbm, acc.at[pl.ds(s * ROWS_PER_TILE, ROWS_PER_TILE)])
    pltpu.sync_copy(ones_hbm, ones_v)
    plsc.subcore_barrier()

    def cp_idx(k, q):
        return pltpu.async_copy(
            edge_hbm.at[1, pl.ds(e0 + k * CHUNK, CHUNK)], cidx[q], isem[q])

    def wait_idx(k, q):
        pltpu.make_async_copy(
            edge_hbm.at[1, pl.ds(e0 + k * CHUNK, CHUNK)],
            cidx[q], isem[q]).wait()

    for q in range(NB):
        cp_idx(q, q)

    def body(i0, carry):
        for q in range(NB):
            i = i0 * NB + q
            wait_idx(i, q)
            pltpu.sync_copy(ones_v, acc.at[cidx[q]], add=True)

            @pl.when(i + NB < N_CHUNKS)
            def _():
                cp_idx(i + NB, q)
        return carry

    lax.fori_loop(0, NFULL // NB, body, 0)
    for i in range(NFULL, N_CHUNKS):
        q = i % NB
        wait_idx(i, q)
        pltpu.sync_copy(ones_v, acc.at[cidx[q]], add=True)

    plsc.subcore_barrier()
    pltpu.sync_copy(
        acc.at[pl.ds(s * ROWS_PER_TILE, ROWS_PER_TILE)],
        out_hbm.at[c, pl.ds(s * ROWS_PER_TILE, ROWS_PER_TILE)],
    )


NI = 2 * NB


def _make_edge_scatter(d):
    @functools.partial(
        pl.kernel,
        out_type=jax.ShapeDtypeStruct((NC, N_PAD, d), jnp.float32),
        mesh=_MESH,
        scratch_types=(
            [pltpu.VMEM((2, CHUNK), jnp.int32) for _ in range(NI)]
            + [pltpu.VMEM((CHUNK, d), jnp.float32) for _ in range(NB)]
            + [pltpu.VMEM_SHARED((N_PAD, d), jnp.float32)]
            + [pltpu.SemaphoreType.DMA for _ in range(NI + 2 * NB)]
        ),
        compiler_params=_SC_PARAMS,
    )
    def edge_kernel(g_hbm, edge_hbm, zeros_hbm, out_hbm, *scr):
        idxb = scr[0:NI]
        rows = scr[NI:NI + NB]
        acc = scr[NI + NB]
        isem = scr[NI + NB + 1:2 * NI + NB + 1]
        gsem = scr[2 * NI + NB + 1:2 * NI + 2 * NB + 1]
        ssem = scr[2 * NI + 2 * NB + 1:2 * NI + 3 * NB + 1]
        c = lax.axis_index("c")
        s = lax.axis_index("s")
        wid = c * NS + s
        e0 = wid * E_PER_W
        pltpu.sync_copy(zeros_hbm,
                        acc.at[pl.ds(s * ROWS_PER_TILE, ROWS_PER_TILE)])
        plsc.subcore_barrier()

        def cp_idx(k, si):
            return pltpu.async_copy(
                edge_hbm.at[:, pl.ds(e0 + k * CHUNK, CHUNK)],
                idxb[si], isem[si])

        def wait_idx(k, si):
            pltpu.make_async_copy(
                edge_hbm.at[:, pl.ds(e0 + k * CHUNK, CHUNK)],
                idxb[si], isem[si]).wait()

        def start_gather(si, q):
            return pltpu.async_copy(g_hbm.at[idxb[si].at[0]], rows[q],
                                    gsem[q])

        def wait_gather(si, q):
            pltpu.make_async_copy(g_hbm.at[idxb[si].at[0]], rows[q],
                                  gsem[q]).wait()

        def start_scatter(si, q):
            return pltpu.async_copy(rows[q], acc.at[idxb[si].at[1]],
                                    ssem[q], add=True)

        def wait_scatter(si, q):
            pltpu.make_async_copy(rows[q], acc.at[idxb[si].at[1]],
                                  ssem[q]).wait()

        def maybe(pred, fn):
            def run():
                fn()
            if isinstance(pred, bool):
                if pred:
                    run()
            else:
                pl.when(pred)(run)

        for si in range(NI - GA):
            cp_idx(si, si)
        for k in range(GA):
            wait_idx(k, k)
            start_gather(k, k)

        def body_one(i, q, si):
            wait_gather(si, q)
            start_scatter(si, q)
            qn = (q + GA) % NB
            sn = (si - (NB - GA)) % NI
            maybe(i >= NB - GA, lambda: wait_scatter(sn, qn))
            maybe(i + NI - GA < N_CHUNKS,
                  lambda: cp_idx(i + NI - GA, (si - GA) % NI))

            def _gather_next():
                wait_idx(i + GA, (si + GA) % NI)
                start_gather((si + GA) % NI, qn)
            maybe(i + GA < N_CHUNKS, _gather_next)

        def outer(j0, carry):
            for k in range(NI):
                body_one(j0 * NI + k, k % NB, k % NI)
            return carry

        NOUTER = N_CHUNKS // NI
        lax.fori_loop(0, NOUTER, outer, 0)
        for i in range(NOUTER * NI, N_CHUNKS):
            body_one(i, i % NB, i % NI)
        for i in range(N_CHUNKS - (NB - GA), N_CHUNKS):
            wait_scatter(i % NI, i % NB)

        plsc.subcore_barrier()
        pltpu.sync_copy(
            acc.at[pl.ds(s * ROWS_PER_TILE, ROWS_PER_TILE)],
            out_hbm.at[c, pl.ds(s * ROWS_PER_TILE, ROWS_PER_TILE)],
        )

    return edge_kernel


_edge_scatter_128 = _make_edge_scatter(D_HID)
_edge_scatter_64 = _make_edge_scatter(D_OUT)


_BLK = 400


def _dis_block(degp_ref):
    deg = degp_ref[0, :, 0:1] + degp_ref[1, :, 0:1]
    return jnp.where(deg > 0, lax.rsqrt(deg), 0.0)


def _pre_body(x_ref, degp_ref, wi_ref, wr_ref, b_ref, g_ref, r_ref):
    dis = _dis_block(degp_ref)
    x = x_ref[...]
    h = jnp.dot(x, wi_ref[...], preferred_element_type=jnp.float32)
    g_ref[...] = h * dis
    r_ref[...] = (
        jnp.dot(x, wr_ref[...], preferred_element_type=jnp.float32)
        + b_ref[...]
    )


def _pre_kernel(x, deg_parts, wi, wr, b):
    d_in, d = wi.shape
    return pl.pallas_call(
        _pre_body,
        grid=(N_NODES // _BLK,),
        in_specs=[
            pl.BlockSpec((_BLK, d_in), lambda i: (i, 0)),
            pl.BlockSpec((NC, _BLK, 16), lambda i: (0, i, 0)),
            pl.BlockSpec((d_in, d), lambda i: (0, 0)),
            pl.BlockSpec((d_in, d), lambda i: (0, 0)),
            pl.BlockSpec((1, d), lambda i: (0, 0)),
        ],
        out_specs=[
            pl.BlockSpec((_BLK, d), lambda i: (i, 0)),
            pl.BlockSpec((_BLK, d), lambda i: (i, 0)),
        ],
        out_shape=[
            jax.ShapeDtypeStruct((N_NODES, d), jnp.float32),
            jax.ShapeDtypeStruct((N_NODES, d), jnp.float32),
        ],
    )(x, deg_parts, wi, wr, b)


def _mm_body(x_ref, wi_ref, wr_ref, b_ref, h_ref, r_ref):
    x = x_ref[...]
    h_ref[...] = jnp.dot(x, wi_ref[...], preferred_element_type=jnp.float32)
    r_ref[...] = (
        jnp.dot(x, wr_ref[...], preferred_element_type=jnp.float32)
        + b_ref[...]
    )


def _mm_kernel(x, wi, wr, b):
    d_in, d = wi.shape
    return pl.pallas_call(
        _mm_body,
        grid=(N_NODES // _BLK,),
        in_specs=[
            pl.BlockSpec((_BLK, d_in), lambda i: (i, 0)),
            pl.BlockSpec((d_in, d), lambda i: (0, 0)),
            pl.BlockSpec((d_in, d), lambda i: (0, 0)),
            pl.BlockSpec((1, d), lambda i: (0, 0)),
        ],
        out_specs=[
            pl.BlockSpec((_BLK, d), lambda i: (i, 0)),
            pl.BlockSpec((_BLK, d), lambda i: (i, 0)),
        ],
        out_shape=[
            jax.ShapeDtypeStruct((N_NODES, d), jnp.float32),
            jax.ShapeDtypeStruct((N_NODES, d), jnp.float32),
        ],
    )(x, wi, wr, b)


def _scale_body(h_ref, degp_ref, g_ref):
    g_ref[...] = h_ref[...] * _dis_block(degp_ref)


def _scale_kernel(h, deg_parts):
    d = h.shape[1]
    return pl.pallas_call(
        _scale_body,
        grid=(N_NODES // _BLK,),
        in_specs=[
            pl.BlockSpec((_BLK, d), lambda i: (i, 0)),
            pl.BlockSpec((NC, _BLK, 16), lambda i: (0, i, 0)),
        ],
        out_specs=pl.BlockSpec((_BLK, d), lambda i: (i, 0)),
        out_shape=jax.ShapeDtypeStruct((N_NODES, d), jnp.float32),
    )(h, deg_parts)


def _mid_body(parts_ref, r_ref, degp_ref, wi_ref, wr_ref, b_ref,
              g_ref, rn_ref):
    dis = _dis_block(degp_ref)
    out = jnp.maximum(
        (parts_ref[0] + parts_ref[1]) * dis + r_ref[...], 0.0)
    h = jnp.dot(out, wi_ref[...], preferred_element_type=jnp.float32)
    g_ref[...] = h * dis
    rn_ref[...] = (
        jnp.dot(out, wr_ref[...], preferred_element_type=jnp.float32)
        + b_ref[...]
    )


def _mid_kernel(parts, r, deg_parts, wi, wr, b):
    d_in, d = wi.shape
    return pl.pallas_call(
        _mid_body,
        grid=(N_NODES // _BLK,),
        in_specs=[
            pl.BlockSpec((NC, _BLK, d_in), lambda i: (0, i, 0)),
            pl.BlockSpec((_BLK, d_in), lambda i: (i, 0)),
            pl.BlockSpec((NC, _BLK, 16), lambda i: (0, i, 0)),
            pl.BlockSpec((d_in, d), lambda i: (0, 0)),
            pl.BlockSpec((d_in, d), lambda i: (0, 0)),
            pl.BlockSpec((1, d), lambda i: (0, 0)),
        ],
        out_specs=[
            pl.BlockSpec((_BLK, d), lambda i: (i, 0)),
            pl.BlockSpec((_BLK, d), lambda i: (i, 0)),
        ],
        out_shape=[
            jax.ShapeDtypeStruct((N_NODES, d), jnp.float32),
            jax.ShapeDtypeStruct((N_NODES, d), jnp.float32),
        ],
    )(parts, r, deg_parts, wi, wr, b)


def _post_body(parts_ref, r_ref, degp_ref, out_ref):
    dis = _dis_block(degp_ref)
    z = (parts_ref[0] + parts_ref[1]) * dis + r_ref[...]
    out_ref[...] = jax.nn.sigmoid(jnp.maximum(z, 0.0))


def _post_kernel(parts, r, deg_parts):
    d = r.shape[1]
    return pl.pallas_call(
        _post_body,
        grid=(N_NODES // _BLK,),
        in_specs=[
            pl.BlockSpec((NC, _BLK, d), lambda i: (0, i, 0)),
            pl.BlockSpec((_BLK, d), lambda i: (i, 0)),
            pl.BlockSpec((NC, _BLK, 16), lambda i: (0, i, 0)),
        ],
        out_specs=pl.BlockSpec((_BLK, d), lambda i: (i, 0)),
        out_shape=jax.ShapeDtypeStruct((N_NODES, d), jnp.float32),
    )(parts, r, deg_parts)


def kernel(x, edge_index, batch, W1_init, W1_root, b1,
           W2_init, W2_root, b2, W3_init, W3_root, b3):
    zeros128 = jnp.zeros((ROWS_PER_TILE, D_HID), jnp.float32)
    zeros64 = jnp.zeros((ROWS_PER_TILE, D_OUT), jnp.float32)
    zeros16 = jnp.zeros((ROWS_PER_TILE, 16), jnp.float32)
    ones16 = jnp.ones((CHUNK, 16), jnp.float32)

    deg_parts = _deg_kernel(edge_index, ones16, zeros16)
    h1, r = _mm_kernel(x, W1_init, W1_root, jnp.reshape(b1, (1, -1)))
    g = _scale_kernel(h1, deg_parts)
    parts = _edge_scatter_128(g, edge_index, zeros128)
    g, r = _mid_kernel(parts, r, deg_parts, W2_init, W2_root,
                       jnp.reshape(b2, (1, -1)))
    parts = _edge_scatter_128(g, edge_index, zeros128)
    g, r = _mid_kernel(parts, r, deg_parts, W3_init, W3_root,
                       jnp.reshape(b3, (1, -1)))
    parts = _edge_scatter_64(g, edge_index, zeros64)
    return _post_kernel(parts, r, deg_parts)


# speedup vs baseline: 3.2427x; 1.0183x over previous
import functools

import jax
import jax.numpy as jnp
from jax import lax
from jax.experimental import pallas as pl
from jax.experimental.pallas import tpu as pltpu
from jax.experimental.pallas import tpu_sc as plsc

N_NODES = 10000
N_PAD = 10240
E = 320000
D_IN = 128
D_HID = 128
D_OUT = 64

NC, NS = 2, 16
NW = NC * NS
E_PER_W = E // NW
CHUNK = 80
N_CHUNKS = E_PER_W // CHUNK
ROWS_PER_TILE = N_PAD // NS

_MESH = plsc.VectorSubcoreMesh(core_axis_name="c", subcore_axis_name="s")
_SC_PARAMS = pltpu.CompilerParams(use_tc_tiling_on_sc=False)


NB = 4
GA = 3
NFULL = (N_CHUNKS // NB) * NB


_DNI = 8
_DDR = 2


@functools.partial(
    pl.kernel,
    out_type=jax.ShapeDtypeStruct((NC, N_PAD, 16), jnp.float32),
    mesh=_MESH,
    scratch_types=(
        [pltpu.VMEM((CHUNK,), jnp.int32) for _ in range(_DNI)]
        + [pltpu.VMEM((CHUNK, 16), jnp.float32),
           pltpu.VMEM_SHARED((N_PAD, 16), jnp.float32)]
        + [pltpu.SemaphoreType.DMA for _ in range(_DNI + NB)]
    ),
    compiler_params=_SC_PARAMS,
)
def _deg_kernel(edge_hbm, ones_hbm, zeros_hbm, out_hbm, *scr):
    cidx = scr[0:_DNI]
    ones_v, acc = scr[_DNI], scr[_DNI + 1]
    isem = scr[_DNI + 2:2 * _DNI + 2]
    ssem = scr[2 * _DNI + 2:2 * _DNI + 2 + NB]
    c = lax.axis_index("c")
    s = lax.axis_index("s")
    wid = c * NS + s
    e0 = wid * E_PER_W
    pltpu.sync_copy(zeros_hbm, acc.at[pl.ds(s * ROWS_PER_TILE, ROWS_PER_TILE)])
    pltpu.sync_copy(ones_hbm, ones_v)
    plsc.subcore_barrier()

    def cp_idx(k, si):
        return pltpu.async_copy(
            edge_hbm.at[1, pl.ds(e0 + k * CHUNK, CHUNK)], cidx[si], isem[si])

    def wait_idx(k, si):
        pltpu.make_async_copy(
            edge_hbm.at[1, pl.ds(e0 + k * CHUNK, CHUNK)],
            cidx[si], isem[si]).wait()

    def start_scatter(si, q):
        return pltpu.async_copy(ones_v, acc.at[cidx[si]], ssem[q], add=True)

    def wait_scatter(si, q):
        pltpu.make_async_copy(ones_v, acc.at[cidx[si]], ssem[q]).wait()

    def maybe(pred, fn):
        def run():
            fn()
        if isinstance(pred, bool):
            if pred:
                run()
        else:
            pl.when(pred)(run)

    for si in range(_DNI - _DDR):
        cp_idx(si, si)

    def body_one(i, q, si):
        wait_idx(i, si)
        start_scatter(si, q)
        maybe(i >= _DDR,
              lambda: wait_scatter((si - _DDR) % _DNI, (q - _DDR) % NB))
        maybe(i + _DNI - _DDR < N_CHUNKS,
              lambda: cp_idx(i + _DNI - _DDR, (si - _DDR) % _DNI))

    def body(i0, carry):
        for k in range(_DNI):
            body_one(i0 * _DNI + k, k % NB, k)
        return carry

    NOUTER = N_CHUNKS // _DNI
    lax.fori_loop(0, NOUTER, body, 0)
    for i in range(NOUTER * _DNI, N_CHUNKS):
        body_one(i, i % NB, i % _DNI)
    for i in range(N_CHUNKS - _DDR, N_CHUNKS):
        wait_scatter(i % _DNI, i % NB)

    plsc.subcore_barrier()
    pltpu.sync_copy(
        acc.at[pl.ds(s * ROWS_PER_TILE, ROWS_PER_TILE)],
        out_hbm.at[c, pl.ds(s * ROWS_PER_TILE, ROWS_PER_TILE)],
    )


NI = 2 * NB


def _make_edge_scatter(d):
    @functools.partial(
        pl.kernel,
        out_type=jax.ShapeDtypeStruct((NC, N_PAD, d), jnp.float32),
        mesh=_MESH,
        scratch_types=(
            [pltpu.VMEM((2, CHUNK), jnp.int32) for _ in range(NI)]
            + [pltpu.VMEM((CHUNK, d), jnp.float32) for _ in range(NB)]
            + [pltpu.VMEM_SHARED((N_PAD, d), jnp.float32)]
            + [pltpu.SemaphoreType.DMA for _ in range(NI + 2 * NB)]
        ),
        compiler_params=_SC_PARAMS,
    )
    def edge_kernel(g_hbm, edge_hbm, zeros_hbm, out_hbm, *scr):
        idxb = scr[0:NI]
        rows = scr[NI:NI + NB]
        acc = scr[NI + NB]
        isem = scr[NI + NB + 1:2 * NI + NB + 1]
        gsem = scr[2 * NI + NB + 1:2 * NI + 2 * NB + 1]
        ssem = scr[2 * NI + 2 * NB + 1:2 * NI + 3 * NB + 1]
        c = lax.axis_index("c")
        s = lax.axis_index("s")
        wid = c * NS + s
        e0 = wid * E_PER_W
        pltpu.sync_copy(zeros_hbm,
                        acc.at[pl.ds(s * ROWS_PER_TILE, ROWS_PER_TILE)])
        plsc.subcore_barrier()

        def cp_idx(k, si):
            return pltpu.async_copy(
                edge_hbm.at[:, pl.ds(e0 + k * CHUNK, CHUNK)],
                idxb[si], isem[si])

        def wait_idx(k, si):
            pltpu.make_async_copy(
                edge_hbm.at[:, pl.ds(e0 + k * CHUNK, CHUNK)],
                idxb[si], isem[si]).wait()

        def start_gather(si, q):
            return pltpu.async_copy(g_hbm.at[idxb[si].at[0]], rows[q],
                                    gsem[q])

        def wait_gather(si, q):
            pltpu.make_async_copy(g_hbm.at[idxb[si].at[0]], rows[q],
                                  gsem[q]).wait()

        def start_scatter(si, q):
            return pltpu.async_copy(rows[q], acc.at[idxb[si].at[1]],
                                    ssem[q], add=True)

        def wait_scatter(si, q):
            pltpu.make_async_copy(rows[q], acc.at[idxb[si].at[1]],
                                  ssem[q]).wait()

        def maybe(pred, fn):
            def run():
                fn()
            if isinstance(pred, bool):
                if pred:
                    run()
            else:
                pl.when(pred)(run)

        for si in range(NI - GA):
            cp_idx(si, si)
        for k in range(GA):
            wait_idx(k, k)
            start_gather(k, k)

        def body_one(i, q, si):
            wait_gather(si, q)
            start_scatter(si, q)
            qn = (q + GA) % NB
            sn = (si - (NB - GA)) % NI
            maybe(i >= NB - GA, lambda: wait_scatter(sn, qn))
            maybe(i + NI - GA < N_CHUNKS,
                  lambda: cp_idx(i + NI - GA, (si - GA) % NI))

            def _gather_next():
                wait_idx(i + GA, (si + GA) % NI)
                start_gather((si + GA) % NI, qn)
            maybe(i + GA < N_CHUNKS, _gather_next)

        def outer(j0, carry):
            for k in range(NI):
                body_one(j0 * NI + k, k % NB, k % NI)
            return carry

        NOUTER = N_CHUNKS // NI
        lax.fori_loop(0, NOUTER, outer, 0)
        for i in range(NOUTER * NI, N_CHUNKS):
            body_one(i, i % NB, i % NI)
        for i in range(N_CHUNKS - (NB - GA), N_CHUNKS):
            wait_scatter(i % NI, i % NB)

        plsc.subcore_barrier()
        pltpu.sync_copy(
            acc.at[pl.ds(s * ROWS_PER_TILE, ROWS_PER_TILE)],
            out_hbm.at[c, pl.ds(s * ROWS_PER_TILE, ROWS_PER_TILE)],
        )

    return edge_kernel


_edge_scatter_128 = _make_edge_scatter(D_HID)
_edge_scatter_64 = _make_edge_scatter(D_OUT)


_BLK = 400


def _dis_block(degp_ref):
    deg = degp_ref[0, :, 0:1] + degp_ref[1, :, 0:1]
    return jnp.where(deg > 0, lax.rsqrt(deg), 0.0)


def _pre_body(x_ref, degp_ref, wi_ref, wr_ref, b_ref, g_ref, r_ref):
    dis = _dis_block(degp_ref)
    x = x_ref[...]
    h = jnp.dot(x, wi_ref[...], preferred_element_type=jnp.float32)
    g_ref[...] = h * dis
    r_ref[...] = (
        jnp.dot(x, wr_ref[...], preferred_element_type=jnp.float32)
        + b_ref[...]
    )


def _pre_kernel(x, deg_parts, wi, wr, b):
    d_in, d = wi.shape
    return pl.pallas_call(
        _pre_body,
        grid=(N_NODES // _BLK,),
        in_specs=[
            pl.BlockSpec((_BLK, d_in), lambda i: (i, 0)),
            pl.BlockSpec((NC, _BLK, 16), lambda i: (0, i, 0)),
            pl.BlockSpec((d_in, d), lambda i: (0, 0)),
            pl.BlockSpec((d_in, d), lambda i: (0, 0)),
            pl.BlockSpec((1, d), lambda i: (0, 0)),
        ],
        out_specs=[
            pl.BlockSpec((_BLK, d), lambda i: (i, 0)),
            pl.BlockSpec((_BLK, d), lambda i: (i, 0)),
        ],
        out_shape=[
            jax.ShapeDtypeStruct((N_NODES, d), jnp.float32),
            jax.ShapeDtypeStruct((N_NODES, d), jnp.float32),
        ],
    )(x, deg_parts, wi, wr, b)


def _mm_body(x_ref, wi_ref, wr_ref, b_ref, h_ref, r_ref):
    x = x_ref[...]
    h_ref[...] = jnp.dot(x, wi_ref[...], preferred_element_type=jnp.float32)
    r_ref[...] = (
        jnp.dot(x, wr_ref[...], preferred_element_type=jnp.float32)
        + b_ref[...]
    )


def _mm_kernel(x, wi, wr, b):
    d_in, d = wi.shape
    return pl.pallas_call(
        _mm_body,
        grid=(N_NODES // _BLK,),
        in_specs=[
            pl.BlockSpec((_BLK, d_in), lambda i: (i, 0)),
            pl.BlockSpec((d_in, d), lambda i: (0, 0)),
            pl.BlockSpec((d_in, d), lambda i: (0, 0)),
            pl.BlockSpec((1, d), lambda i: (0, 0)),
        ],
        out_specs=[
            pl.BlockSpec((_BLK, d), lambda i: (i, 0)),
            pl.BlockSpec((_BLK, d), lambda i: (i, 0)),
        ],
        out_shape=[
            jax.ShapeDtypeStruct((N_NODES, d), jnp.float32),
            jax.ShapeDtypeStruct((N_NODES, d), jnp.float32),
        ],
    )(x, wi, wr, b)


def _scale_body(h_ref, degp_ref, g_ref):
    g_ref[...] = h_ref[...] * _dis_block(degp_ref)


def _scale_kernel(h, deg_parts):
    d = h.shape[1]
    return pl.pallas_call(
        _scale_body,
        grid=(N_NODES // _BLK,),
        in_specs=[
            pl.BlockSpec((_BLK, d), lambda i: (i, 0)),
            pl.BlockSpec((NC, _BLK, 16), lambda i: (0, i, 0)),
        ],
        out_specs=pl.BlockSpec((_BLK, d), lambda i: (i, 0)),
        out_shape=jax.ShapeDtypeStruct((N_NODES, d), jnp.float32),
    )(h, deg_parts)


def _mid_body(parts_ref, r_ref, degp_ref, wi_ref, wr_ref, b_ref,
              g_ref, rn_ref):
    dis = _dis_block(degp_ref)
    out = jnp.maximum(
        (parts_ref[0] + parts_ref[1]) * dis + r_ref[...], 0.0)
    h = jnp.dot(out, wi_ref[...], preferred_element_type=jnp.float32)
    g_ref[...] = h * dis
    rn_ref[...] = (
        jnp.dot(out, wr_ref[...], preferred_element_type=jnp.float32)
        + b_ref[...]
    )


def _mid_kernel(parts, r, deg_parts, wi, wr, b):
    d_in, d = wi.shape
    return pl.pallas_call(
        _mid_body,
        grid=(N_NODES // _BLK,),
        in_specs=[
            pl.BlockSpec((NC, _BLK, d_in), lambda i: (0, i, 0)),
            pl.BlockSpec((_BLK, d_in), lambda i: (i, 0)),
            pl.BlockSpec((NC, _BLK, 16), lambda i: (0, i, 0)),
            pl.BlockSpec((d_in, d), lambda i: (0, 0)),
            pl.BlockSpec((d_in, d), lambda i: (0, 0)),
            pl.BlockSpec((1, d), lambda i: (0, 0)),
        ],
        out_specs=[
            pl.BlockSpec((_BLK, d), lambda i: (i, 0)),
            pl.BlockSpec((_BLK, d), lambda i: (i, 0)),
        ],
        out_shape=[
            jax.ShapeDtypeStruct((N_NODES, d), jnp.float32),
            jax.ShapeDtypeStruct((N_NODES, d), jnp.float32),
        ],
    )(parts, r, deg_parts, wi, wr, b)


def _post_body(parts_ref, r_ref, degp_ref, out_ref):
    dis = _dis_block(degp_ref)
    z = (parts_ref[0] + parts_ref[1]) * dis + r_ref[...]
    out_ref[...] = jax.nn.sigmoid(jnp.maximum(z, 0.0))


def _post_kernel(parts, r, deg_parts):
    d = r.shape[1]
    return pl.pallas_call(
        _post_body,
        grid=(N_NODES // _BLK,),
        in_specs=[
            pl.BlockSpec((NC, _BLK, d), lambda i: (0, i, 0)),
            pl.BlockSpec((_BLK, d), lambda i: (i, 0)),
            pl.BlockSpec((NC, _BLK, 16), lambda i: (0, i, 0)),
        ],
        out_specs=pl.BlockSpec((_BLK, d), lambda i: (i, 0)),
        out_shape=jax.ShapeDtypeStruct((N_NODES, d), jnp.float32),
    )(parts, r, deg_parts)


def kernel(x, edge_index, batch, W1_init, W1_root, b1,
           W2_init, W2_root, b2, W3_init, W3_root, b3):
    zeros128 = jnp.zeros((ROWS_PER_TILE, D_HID), jnp.float32)
    zeros64 = jnp.zeros((ROWS_PER_TILE, D_OUT), jnp.float32)
    zeros16 = jnp.zeros((ROWS_PER_TILE, 16), jnp.float32)
    ones16 = jnp.ones((CHUNK, 16), jnp.float32)

    deg_parts = _deg_kernel(edge_index, ones16, zeros16)
    h1, r = _mm_kernel(x, W1_init, W1_root, jnp.reshape(b1, (1, -1)))
    g = _scale_kernel(h1, deg_parts)
    parts = _edge_scatter_128(g, edge_index, zeros128)
    g, r = _mid_kernel(parts, r, deg_parts, W2_init, W2_root,
                       jnp.reshape(b2, (1, -1)))
    parts = _edge_scatter_128(g, edge_index, zeros128)
    g, r = _mid_kernel(parts, r, deg_parts, W3_init, W3_root,
                       jnp.reshape(b3, (1, -1)))
    parts = _edge_scatter_64(g, edge_index, zeros64)
    return _post_kernel(parts, r, deg_parts)
